# baseline (speedup 1.0000x reference)
import functools

import jax
import jax.numpy as jnp
from jax.experimental import pallas as pl
from jax.experimental.pallas import tpu as pltpu
from jax.experimental.pallas import tpu_sc as plsc

B = 256
C_IN = 32
T0 = 2048
U = 512
HIDDEN = 64
OUT = 32
KNN = 8


def _mm(a, w):
    return jax.lax.dot_general(a, w, (((1,), (0,)), ((), ())),
                               preferred_element_type=jnp.float32)


def _conv_body(xph_ref, w1_ref, b1_ref, w2_ref, b2_ref, out_ref):
    ph = [xph_ref[0, j] for j in range(4)]
    iota = jax.lax.broadcasted_iota(jnp.int32, (U, 1), 0)

    def up(a):
        return jnp.where(iota == 0, 0.0, jnp.roll(a, 1, axis=0))

    def dn(a):
        return jnp.where(iota == U - 1, 0.0, jnp.roll(a, -1, axis=0))

    w1 = [w1_ref[dt] for dt in range(5)]
    b1 = b1_ref[0]
    out1 = []
    for j in range(4):
        acc = None
        for dt in range(5):
            s = j + dt - 2
            if s < 0:
                a = up(ph[s + 4])
            elif s >= 4:
                a = dn(ph[s - 4])
            else:
                a = ph[s]
            t = _mm(a, w1[dt])
            acc = t if acc is None else acc + t
        out1.append(jnp.maximum(acc + b1, 0.0))
    p1 = [jnp.maximum(out1[0], out1[1]), jnp.maximum(out1[2], out1[3])]

    w2 = [w2_ref[dt] for dt in range(5)]
    b2 = b2_ref[0]
    seq0 = [up(p1[0]), up(p1[1]), p1[0], p1[1], dn(p1[0])]
    seq1 = [up(p1[1]), p1[0], p1[1], dn(p1[0]), dn(p1[1])]
    o20 = b2
    o21 = b2
    for dt in range(5):
        o20 = o20 + _mm(seq0[dt], w2[dt])
        o21 = o21 + _mm(seq1[dt], w2[dt])
    out_ref[0] = jnp.maximum(jnp.maximum(o20, 0.0), jnp.maximum(o21, 0.0))


def _graph_body(h0_ref, idx_ref):
    h0 = h0_ref[...]
    g = jax.lax.dot_general(h0, h0, (((1,), (1,)), ((), ())),
                            preferred_element_type=jnp.float32)
    sqc = jnp.sum(h0 * h0, axis=1, keepdims=True)
    sqr = jnp.sum(h0 * h0, axis=1)[None, :]
    row = jax.lax.broadcasted_iota(jnp.int32, (U, U), 0)
    col = jax.lax.broadcasted_iota(jnp.int32, (U, U), 1)
    eye = row == col
    d2 = sqc + sqr - 2.0 * g + jnp.where(eye, 1e9, 0.0)
    col16 = jax.lax.broadcasted_iota(jnp.int32, (U, 16), 1)
    row16 = jax.lax.broadcasted_iota(jnp.int32, (U, 16), 0)
    acc16 = jnp.where(col16 == KNN, row16, 0)
    for k in range(KNN):
        m = jnp.min(d2, axis=1, keepdims=True)
        cand = jnp.where(d2 == m, col, U)
        idx = jnp.min(cand, axis=1, keepdims=True)
        sel = col == idx
        acc16 = jnp.where(col16 == k, idx, acc16)
        d2 = jnp.where(sel, 3e9, d2)
    idx_ref[...] = acc16


_SC_NC = 2
_SC_NS = 16
_ROWS_PER_W = U // (_SC_NC * _SC_NS)


def _build_m_sc(idx16):
    mesh = plsc.VectorSubcoreMesh(core_axis_name="c", subcore_axis_name="s")

    @functools.partial(
        pl.kernel,
        mesh=mesh,
        out_type=jax.ShapeDtypeStruct((U, U), jnp.float32),
        scratch_types=[
            pltpu.VMEM((_ROWS_PER_W, 16), jnp.int32),
            pltpu.VMEM((U,), jnp.float32),
        ],
        compiler_params=pltpu.CompilerParams(needs_layout_passes=False),
    )
    def mbuild(idx_hbm, m_hbm, idx_v, row_v):
        c = jax.lax.axis_index("c")
        s = jax.lax.axis_index("s")
        base = (s * _SC_NC + c) * _ROWS_PER_W
        pltpu.sync_copy(idx_hbm.at[pl.ds(base, _ROWS_PER_W)], idx_v)
        lane = jax.lax.broadcasted_iota(jnp.int32, (16,), 0)
        mask = lane < KNN + 1
        vals = jnp.full((16,), 1.0 / 9.0, dtype=jnp.float32)
        zeros16 = jnp.zeros((16,), jnp.float32)
        for r in range(_ROWS_PER_W):
            for z in range(U // 16):
                row_v[pl.ds(z * 16, 16)] = zeros16
            plsc.store_scatter(row_v, [idx_v[r]], vals, mask=mask)
            pltpu.sync_copy(row_v, m_hbm.at[base + r])

    return mbuild(idx16)


GB = 8


def _gcn_body(m_ref, h_ref, w1_ref, b1_ref, w2_ref, b2_ref,
              fcw_ref, fcb_ref, out_ref):
    mop = m_ref[...]
    pooled = []
    for i in range(GB):
        h = h_ref[i]
        a1 = jnp.maximum(_mm(mop, _mm(h, w1_ref[...])) + b1_ref[0], 0.0)
        a2 = jnp.maximum(_mm(mop, _mm(a1, w2_ref[...])) + b2_ref[0], 0.0)
        pooled.append(jnp.sum(a2, axis=0, keepdims=True) * (1.0 / U))
    pooled = jnp.concatenate(pooled, axis=0)
    out_ref[...] = _mm(pooled, fcw_ref[...]) + fcb_ref[0]


def kernel(x, conv1_w, conv1_b, conv2_w, conv2_b,
           gcn1_w, gcn1_b, gcn2_w, gcn2_b, fc_w, fc_b):
    xph = jnp.transpose(x, (0, 2, 1)).reshape(B, U, 4, C_IN)
    xph = jnp.transpose(xph, (0, 2, 1, 3))
    w1t = jnp.transpose(conv1_w, (2, 1, 0))
    w2t = jnp.transpose(conv2_w, (2, 1, 0))
    fcwt = jnp.transpose(fc_w, (1, 0))
    b1 = conv1_b[None, :]
    b2 = conv2_b[None, :]
    g1b = gcn1_b[None, :]
    g2b = gcn2_b[None, :]
    fcb = fc_b[None, :]

    h = pl.pallas_call(
        _conv_body,
        grid=(B,),
        in_specs=[
            pl.BlockSpec((1, 4, U, C_IN), lambda b: (b, 0, 0, 0)),
            pl.BlockSpec((5, C_IN, 16), lambda b: (0, 0, 0)),
            pl.BlockSpec((1, 16), lambda b: (0, 0)),
            pl.BlockSpec((5, 16, C_IN), lambda b: (0, 0, 0)),
            pl.BlockSpec((1, C_IN), lambda b: (0, 0)),
        ],
        out_specs=pl.BlockSpec((1, U, C_IN), lambda b: (b, 0, 0)),
        out_shape=jax.ShapeDtypeStruct((B, U, C_IN), jnp.float32),
    )(xph, w1t, b1, w2t, b2)

    idx16 = pl.pallas_call(
        _graph_body,
        out_shape=jax.ShapeDtypeStruct((U, 16), jnp.int32),
    )(h[0])
    mop = _build_m_sc(idx16)

    out = pl.pallas_call(
        _gcn_body,
        grid=(B // GB,),
        in_specs=[
            pl.BlockSpec((U, U), lambda b: (0, 0)),
            pl.BlockSpec((GB, U, C_IN), lambda b: (b, 0, 0)),
            pl.BlockSpec((C_IN, HIDDEN), lambda b: (0, 0)),
            pl.BlockSpec((1, HIDDEN), lambda b: (0, 0)),
            pl.BlockSpec((HIDDEN, HIDDEN), lambda b: (0, 0)),
            pl.BlockSpec((1, HIDDEN), lambda b: (0, 0)),
            pl.BlockSpec((HIDDEN, OUT), lambda b: (0, 0)),
            pl.BlockSpec((1, OUT), lambda b: (0, 0)),
        ],
        out_specs=pl.BlockSpec((GB, OUT), lambda b: (b, 0)),
        out_shape=jax.ShapeDtypeStruct((B, OUT), jnp.float32),
    )(mop, h, gcn1_w, g1b, gcn2_w, g2b, fcwt, fcb)
    return out


# speedup vs baseline: 1.9046x; 1.9046x over previous
import functools

import jax
import jax.numpy as jnp
from jax.experimental import pallas as pl
from jax.experimental.pallas import tpu as pltpu
from jax.experimental.pallas import tpu_sc as plsc

B = 256
C_IN = 32
T0 = 2048
U = 512
HIDDEN = 64
OUT = 32
KNN = 8

CB = 4
GL = 4
GB = 32


def _mm(a, w):
    return jax.lax.dot_general(a, w, (((1,), (0,)), ((), ())),
                               preferred_element_type=jnp.float32)


def _conv_body(x_ref, w1lo_ref, w1mid_ref, w1hi_ref, b1_ref,
               w2lo_ref, w2mid_ref, w2hi_ref, b2_ref, out_ref):
    iota = jax.lax.broadcasted_iota(jnp.int32, (U, 1), 0)

    def up(a):
        return jnp.where(iota == 0, 0.0, jnp.roll(a, 1, axis=0))

    def dn(a):
        return jnp.where(iota == U - 1, 0.0, jnp.roll(a, -1, axis=0))

    for s in range(CB):
        xc = x_ref[s]
        o1 = (_mm(xc, w1mid_ref[...])
              + _mm(up(xc[:, 64:128]), w1lo_ref[...])
              + _mm(dn(xc[:, 0:64]), w1hi_ref[...])
              + b1_ref[0])
        a = jnp.maximum(o1, 0.0)
        p1 = jnp.concatenate(
            [jnp.maximum(a[:, 0:16], a[:, 16:32]),
             jnp.maximum(a[:, 32:48], a[:, 48:64])], axis=1)
        o2 = (_mm(p1, w2mid_ref[...])
              + _mm(up(p1), w2lo_ref[...])
              + _mm(dn(p1), w2hi_ref[...])
              + b2_ref[0])
        r = jnp.maximum(o2, 0.0)
        out_ref[s] = jnp.maximum(r[:, 0:32], r[:, 32:64])


def _graph_body(h0_ref, idx_ref):
    h0 = h0_ref[...]
    g = jax.lax.dot_general(h0, h0, (((1,), (1,)), ((), ())),
                            preferred_element_type=jnp.float32)
    sqc = jnp.sum(h0 * h0, axis=1, keepdims=True)
    sqr = jnp.sum(h0 * h0, axis=1)[None, :]
    row = jax.lax.broadcasted_iota(jnp.int32, (U, U), 0)
    col = jax.lax.broadcasted_iota(jnp.int32, (U, U), 1)
    eye = row == col
    d2 = sqc + sqr - 2.0 * g + jnp.where(eye, 1e9, 0.0)
    col16 = jax.lax.broadcasted_iota(jnp.int32, (U, 16), 1)
    row16 = jax.lax.broadcasted_iota(jnp.int32, (U, 16), 0)
    acc16 = jnp.where(col16 == KNN, row16, 0)
    for k in range(KNN):
        m = jnp.min(d2, axis=1, keepdims=True)
        cand = jnp.where(d2 == m, col, U)
        idx = jnp.min(cand, axis=1, keepdims=True)
        sel = col == idx
        acc16 = jnp.where(col16 == k, idx, acc16)
        d2 = jnp.where(sel, 3e9, d2)
    idx_ref[...] = acc16


_SC_NC = 2
_SC_NS = 16
_ROWS_PER_W = U // (_SC_NC * _SC_NS)


def _build_m_sc(idx16):
    mesh = plsc.VectorSubcoreMesh(core_axis_name="c", subcore_axis_name="s")

    @functools.partial(
        pl.kernel,
        mesh=mesh,
        out_type=jax.ShapeDtypeStruct((U, U), jnp.float32),
        scratch_types=[
            pltpu.VMEM((_ROWS_PER_W, 16), jnp.int32),
            pltpu.VMEM((U,), jnp.float32),
        ],
        compiler_params=pltpu.CompilerParams(needs_layout_passes=False),
    )
    def mbuild(idx_hbm, m_hbm, idx_v, row_v):
        c = jax.lax.axis_index("c")
        s = jax.lax.axis_index("s")
        base = (s * _SC_NC + c) * _ROWS_PER_W
        pltpu.sync_copy(idx_hbm.at[pl.ds(base, _ROWS_PER_W)], idx_v)
        lane = jax.lax.broadcasted_iota(jnp.int32, (16,), 0)
        mask = lane < KNN + 1
        vals = jnp.full((16,), 1.0 / 9.0, dtype=jnp.float32)
        zeros16 = jnp.zeros((16,), jnp.float32)
        for r in range(_ROWS_PER_W):
            for z in range(U // 16):
                row_v[pl.ds(z * 16, 16)] = zeros16
            plsc.store_scatter(row_v, [idx_v[r]], vals, mask=mask)
            pltpu.sync_copy(row_v, m_hbm.at[base + r])

    return mbuild(idx16)


def _gcn_body(m_ref, h_ref, w1_ref, b1_ref, w2_ref, b2_ref,
              fcw_ref, fcb_ref, out_ref):
    mop = m_ref[...]
    pooled = []
    for g in range(GB // GL):
        h4 = jnp.concatenate([h_ref[GL * g + i] for i in range(GL)],
                             axis=1)
        hw = _mm(h4, w1_ref[...])
        a1 = jnp.maximum(_mm(mop, hw) + b1_ref[0], 0.0)
        a2 = jnp.maximum(_mm(mop, _mm(a1, w2_ref[...])) + b2_ref[0], 0.0)
        pooled.append(jnp.sum(a2, axis=0, keepdims=True) * (1.0 / U))
    pooled = jnp.concatenate(pooled, axis=0)
    out_ref[...] = _mm(pooled, fcw_ref[...]) + fcb_ref[0]


def kernel(x, conv1_w, conv1_b, conv2_w, conv2_b,
           gcn1_w, gcn1_b, gcn2_w, gcn2_b, fc_w, fc_b):
    f32 = jnp.float32
    xcat = jnp.transpose(x.reshape(B, C_IN, U, 4), (0, 2, 3, 1)) \
              .reshape(B, U, 4 * C_IN)

    w1all = jnp.zeros((256, 64), f32)
    for j in range(4):
        for dt in range(5):
            blk = j + dt
            w1all = w1all.at[32 * blk:32 * (blk + 1),
                             16 * j:16 * (j + 1)].set(conv1_w[:, :, dt].T)
    w1lo, w1mid, w1hi = w1all[0:64], w1all[64:192], w1all[192:256]
    w2all = jnp.zeros((96, 64), f32)
    for j in range(2):
        for dt in range(5):
            blk = j + dt
            w2all = w2all.at[16 * blk:16 * (blk + 1),
                             32 * j:32 * (j + 1)].set(conv2_w[:, :, dt].T)
    w2lo, w2mid, w2hi = w2all[0:32], w2all[32:64], w2all[64:96]
    b1c = jnp.tile(conv1_b, 4)[None, :]
    b2c = jnp.tile(conv2_b, 2)[None, :]

    w1blk = jnp.zeros((GL * C_IN, GL * HIDDEN), f32)
    w2blk = jnp.zeros((GL * HIDDEN, GL * HIDDEN), f32)
    fcblk = jnp.zeros((GL * HIDDEN, GL * OUT), f32)
    for g in range(GL):
        w1blk = w1blk.at[C_IN * g:C_IN * (g + 1),
                         HIDDEN * g:HIDDEN * (g + 1)].set(gcn1_w)
        w2blk = w2blk.at[HIDDEN * g:HIDDEN * (g + 1),
                         HIDDEN * g:HIDDEN * (g + 1)].set(gcn2_w)
        fcblk = fcblk.at[HIDDEN * g:HIDDEN * (g + 1),
                         OUT * g:OUT * (g + 1)].set(fc_w.T)
    g1bt = jnp.tile(gcn1_b, GL)[None, :]
    g2bt = jnp.tile(gcn2_b, GL)[None, :]
    fcbt = jnp.tile(fc_b, GL)[None, :]

    h = pl.pallas_call(
        _conv_body,
        grid=(B // CB,),
        in_specs=[
            pl.BlockSpec((CB, U, 4 * C_IN), lambda b: (b, 0, 0)),
            pl.BlockSpec((64, 64), lambda b: (0, 0)),
            pl.BlockSpec((128, 64), lambda b: (0, 0)),
            pl.BlockSpec((64, 64), lambda b: (0, 0)),
            pl.BlockSpec((1, 64), lambda b: (0, 0)),
            pl.BlockSpec((32, 64), lambda b: (0, 0)),
            pl.BlockSpec((32, 64), lambda b: (0, 0)),
            pl.BlockSpec((32, 64), lambda b: (0, 0)),
            pl.BlockSpec((1, 64), lambda b: (0, 0)),
        ],
        out_specs=pl.BlockSpec((CB, U, C_IN), lambda b: (b, 0, 0)),
        out_shape=jax.ShapeDtypeStruct((B, U, C_IN), jnp.float32),
    )(xcat, w1lo, w1mid, w1hi, b1c, w2lo, w2mid, w2hi, b2c)

    idx16 = pl.pallas_call(
        _graph_body,
        out_shape=jax.ShapeDtypeStruct((U, 16), jnp.int32),
    )(h[0])
    mop = _build_m_sc(idx16)

    out = pl.pallas_call(
        _gcn_body,
        grid=(B // GB,),
        in_specs=[
            pl.BlockSpec((U, U), lambda b: (0, 0)),
            pl.BlockSpec((GB, U, C_IN), lambda b: (b, 0, 0)),
            pl.BlockSpec((GL * C_IN, GL * HIDDEN), lambda b: (0, 0)),
            pl.BlockSpec((1, GL * HIDDEN), lambda b: (0, 0)),
            pl.BlockSpec((GL * HIDDEN, GL * HIDDEN), lambda b: (0, 0)),
            pl.BlockSpec((1, GL * HIDDEN), lambda b: (0, 0)),
            pl.BlockSpec((GL * HIDDEN, GL * OUT), lambda b: (0, 0)),
            pl.BlockSpec((1, GL * OUT), lambda b: (0, 0)),
        ],
        out_specs=pl.BlockSpec((GB // GL, GL * OUT), lambda b: (b, 0)),
        out_shape=jax.ShapeDtypeStruct((B // GL, GL * OUT), jnp.float32),
    )(mop, h, w1blk, g1bt, w2blk, g2bt, fcblk, fcbt)
    return out.reshape(B, OUT)
